# baseline (speedup 1.0000x reference)
import functools
import math

import jax
import jax.numpy as jnp
from jax import lax
from jax.experimental import pallas as pl
from jax.experimental.pallas import tpu as pltpu
from jax.experimental.pallas import tpu_sc as plsc

_B = 4096
_R = 64
_NFF = 1024
_SCALE = math.sqrt(2.0 / _NFF)

_NC = 2
_NS = 16
_NW = _NC * _NS
_BPW = _B // _NW


_L = 16


def _sc_gather_body(idx0_hbm, idx1_hbm, idx2_hbm, u0_hbm, u1_hbm, u2_hbm,
                    g0_hbm, g1_hbm, g2_hbm,
                    idx0_v, idx1_v, idx2_v, sem0, sem1, sem2):
    wid = lax.axis_index("s") * _NC + lax.axis_index("c")
    base = wid * _BPW
    pltpu.sync_copy(idx0_hbm.at[pl.ds(base, _BPW)], idx0_v)
    pltpu.sync_copy(idx1_hbm.at[pl.ds(base, _BPW)], idx1_v)
    pltpu.sync_copy(idx2_hbm.at[pl.ds(base, _BPW)], idx2_v)
    lane = lax.iota(jnp.int32, _L)

    def group(g, _):
        v0 = idx0_v[pl.ds(g * _L, _L)]
        v1 = idx1_v[pl.ds(g * _L, _L)]
        v2 = idx2_v[pl.ds(g * _L, _L)]
        for j in range(_L):
            r = base + g * _L + j
            m = lane == j
            i0 = jnp.sum(jnp.where(m, v0, 0))
            i1 = jnp.sum(jnp.where(m, v1, 0))
            i2 = jnp.sum(jnp.where(m, v2, 0))
            pltpu.async_copy(u0_hbm.at[pl.ds(i0, 1)], g0_hbm.at[pl.ds(r, 1)], sem0)
            pltpu.async_copy(u1_hbm.at[pl.ds(i1, 1)], g1_hbm.at[pl.ds(r, 1)], sem1)
            pltpu.async_copy(u2_hbm.at[pl.ds(i2, 1)], g2_hbm.at[pl.ds(r, 1)], sem2)
        return 0

    lax.fori_loop(0, _BPW // _L, group, 0)
    pltpu.make_async_copy(u0_hbm.at[pl.ds(0, _BPW)], g0_hbm.at[pl.ds(base, _BPW)], sem0).wait()
    pltpu.make_async_copy(u1_hbm.at[pl.ds(0, _BPW)], g1_hbm.at[pl.ds(base, _BPW)], sem1).wait()
    pltpu.make_async_copy(u2_hbm.at[pl.ds(0, _BPW)], g2_hbm.at[pl.ds(base, _BPW)], sem2).wait()


@functools.cache
def _sc_gather():
    return pl.kernel(
        _sc_gather_body,
        out_type=(
            jax.ShapeDtypeStruct((_B, _R), jnp.float32),
            jax.ShapeDtypeStruct((_B, _R), jnp.float32),
            jax.ShapeDtypeStruct((_B, _R), jnp.float32),
        ),
        mesh=plsc.VectorSubcoreMesh(core_axis_name="c", subcore_axis_name="s"),
        scratch_types=[
            pltpu.VMEM((_BPW,), jnp.int32),
            pltpu.VMEM((_BPW,), jnp.int32),
            pltpu.VMEM((_BPW,), jnp.int32),
            pltpu.SemaphoreType.DMA,
            pltpu.SemaphoreType.DMA,
            pltpu.SemaphoreType.DMA,
        ],
        compiler_params=pltpu.CompilerParams(needs_layout_passes=False),
    )


def _mlp_body(g0, g1, g2, t, w0, w1, w2, wt, bff, wout, bout, out):
    acc = jnp.dot(g0[...], w0[...], preferred_element_type=jnp.float32)
    acc += jnp.dot(g1[...], w1[...], preferred_element_type=jnp.float32)
    acc += jnp.dot(g2[...], w2[...], preferred_element_type=jnp.float32)
    acc += t[...] * wt[...]
    acc += bff[...]
    feat = jnp.cos(acc) * _SCALE
    out[...] = jnp.dot(feat, wout[...], preferred_element_type=jnp.float32) + bout[...]


def _mlp_call(blk):
    grid = _B // blk
    const = lambda shape: pl.BlockSpec(shape, lambda i: (0, 0))
    return pl.pallas_call(
        _mlp_body,
        grid=(grid,),
        in_specs=[
            pl.BlockSpec((blk, _R), lambda i: (i, 0)),
            pl.BlockSpec((blk, _R), lambda i: (i, 0)),
            pl.BlockSpec((blk, _R), lambda i: (i, 0)),
            pl.BlockSpec((blk, 1), lambda i: (i, 0)),
            const((_R, _NFF)),
            const((_R, _NFF)),
            const((_R, _NFF)),
            const((1, _NFF)),
            const((1, _NFF)),
            const((_NFF, 1)),
            const((1, 1)),
        ],
        out_specs=pl.BlockSpec((blk, 1), lambda i: (i, 0)),
        out_shape=jax.ShapeDtypeStruct((_B, 1), jnp.float32),
    )


@jax.jit
def kernel(b_i_n, b_t_n, U0, U1, U2, W_ff, b_ff, W_out, b_out):
    idx0 = b_i_n[:, 0]
    idx1 = b_i_n[:, 1]
    idx2 = b_i_n[:, 2]
    g0, g1, g2 = _sc_gather()(idx0, idx1, idx2, U0, U1, U2)
    w0 = W_ff[0:_R]
    w1 = W_ff[_R:2 * _R]
    w2 = W_ff[2 * _R:3 * _R]
    wt = W_ff[3 * _R:3 * _R + 1]
    y = _mlp_call(512)(
        g0, g1, g2, b_t_n.reshape(_B, 1),
        w0, w1, w2, wt, b_ff.reshape(1, _NFF),
        W_out, b_out.reshape(1, 1),
    )
    return y


# speedup vs baseline: 1.5745x; 1.5745x over previous
import functools
import math

import jax
import jax.numpy as jnp
from jax import lax
from jax.experimental import pallas as pl
from jax.experimental.pallas import tpu as pltpu
from jax.experimental.pallas import tpu_sc as plsc

_B = 4096
_R = 64
_NFF = 1024
_SCALE = math.sqrt(2.0 / _NFF)

_NC = 2
_NS = 16
_NW = _NC * _NS
_BPW = _B // _NW


_RP = 128


def _sc_gather_body(idx0_hbm, idx1_hbm, idx2_hbm, u0_hbm, u1_hbm, u2_hbm,
                    g0_hbm, g1_hbm, g2_hbm,
                    idx0_v, idx1_v, idx2_v, r0_v, r1_v, r2_v,
                    sem0, sem1, sem2):
    wid = lax.axis_index("s") * _NC + lax.axis_index("c")
    base = wid * _BPW
    pltpu.sync_copy(idx0_hbm.at[pl.ds(base, _BPW)], idx0_v)
    pltpu.sync_copy(idx1_hbm.at[pl.ds(base, _BPW)], idx1_v)
    pltpu.sync_copy(idx2_hbm.at[pl.ds(base, _BPW)], idx2_v)
    c0 = pltpu.async_copy(u0_hbm.at[idx0_v], r0_v, sem0)
    c1 = pltpu.async_copy(u1_hbm.at[idx1_v], r1_v, sem1)
    c2 = pltpu.async_copy(u2_hbm.at[idx2_v], r2_v, sem2)
    c0.wait()
    pltpu.sync_copy(r0_v, g0_hbm.at[pl.ds(base, _BPW)])
    c1.wait()
    pltpu.sync_copy(r1_v, g1_hbm.at[pl.ds(base, _BPW)])
    c2.wait()
    pltpu.sync_copy(r2_v, g2_hbm.at[pl.ds(base, _BPW)])


@functools.cache
def _sc_gather():
    return pl.kernel(
        _sc_gather_body,
        out_type=(
            jax.ShapeDtypeStruct((_B, _RP), jnp.float32),
            jax.ShapeDtypeStruct((_B, _RP), jnp.float32),
            jax.ShapeDtypeStruct((_B, _RP), jnp.float32),
        ),
        mesh=plsc.VectorSubcoreMesh(core_axis_name="c", subcore_axis_name="s"),
        scratch_types=[
            pltpu.VMEM((_BPW,), jnp.int32),
            pltpu.VMEM((_BPW,), jnp.int32),
            pltpu.VMEM((_BPW,), jnp.int32),
            pltpu.VMEM((_BPW, _RP), jnp.float32),
            pltpu.VMEM((_BPW, _RP), jnp.float32),
            pltpu.VMEM((_BPW, _RP), jnp.float32),
            pltpu.SemaphoreType.DMA,
            pltpu.SemaphoreType.DMA,
            pltpu.SemaphoreType.DMA,
        ],
    )


def _mlp_body(g0, g1, g2, t, w0, w1, w2, wt, bff, wout, bout, out):
    acc = jnp.dot(g0[...], w0[...], preferred_element_type=jnp.float32)
    acc += jnp.dot(g1[...], w1[...], preferred_element_type=jnp.float32)
    acc += jnp.dot(g2[...], w2[...], preferred_element_type=jnp.float32)
    acc += t[...] * wt[...]
    acc += bff[...]
    feat = jnp.cos(acc) * _SCALE
    out[...] = jnp.dot(feat, wout[...], preferred_element_type=jnp.float32) + bout[...]


def _mlp_call(blk):
    grid = _B // blk
    const = lambda shape: pl.BlockSpec(shape, lambda i: (0, 0))
    return pl.pallas_call(
        _mlp_body,
        grid=(grid,),
        in_specs=[
            pl.BlockSpec((blk, _RP), lambda i: (i, 0)),
            pl.BlockSpec((blk, _RP), lambda i: (i, 0)),
            pl.BlockSpec((blk, _RP), lambda i: (i, 0)),
            pl.BlockSpec((blk, 1), lambda i: (i, 0)),
            const((_RP, _NFF)),
            const((_RP, _NFF)),
            const((_RP, _NFF)),
            const((1, _NFF)),
            const((1, _NFF)),
            const((_NFF, 1)),
            const((1, 1)),
        ],
        out_specs=pl.BlockSpec((blk, 1), lambda i: (i, 0)),
        out_shape=jax.ShapeDtypeStruct((_B, 1), jnp.float32),
    )


@jax.jit
def kernel(b_i_n, b_t_n, U0, U1, U2, W_ff, b_ff, W_out, b_out):
    idx0 = b_i_n[:, 0]
    idx1 = b_i_n[:, 1]
    idx2 = b_i_n[:, 2]
    pad = ((0, 0), (0, _RP - _R))
    u0p = jnp.pad(U0, pad)
    u1p = jnp.pad(U1, pad)
    u2p = jnp.pad(U2, pad)
    g0, g1, g2 = _sc_gather()(idx0, idx1, idx2, u0p, u1p, u2p)
    wpad = ((0, _RP - _R), (0, 0))
    w0 = jnp.pad(W_ff[0:_R], wpad)
    w1 = jnp.pad(W_ff[_R:2 * _R], wpad)
    w2 = jnp.pad(W_ff[2 * _R:3 * _R], wpad)
    wt = W_ff[3 * _R:3 * _R + 1]
    y = _mlp_call(512)(
        g0, g1, g2, b_t_n.reshape(_B, 1),
        w0, w1, w2, wt, b_ff.reshape(1, _NFF),
        W_out, b_out.reshape(1, 1),
    )
    return y
